# baseline (speedup 1.0000x reference)
import functools

import jax
import jax.numpy as jnp
from jax import lax
from jax.experimental import pallas as pl
from jax.experimental.pallas import tpu as pltpu
from jax.experimental.pallas import tpu_sc as plsc

N = 10000
E = 320000
F = 128
NH = 8
HD = 16
NG = 2

SELU_ALPHA = 1.6732632423543772
SELU_SCALE = 1.0507009873554805

NC = 2
NS = 16
EPW = E // NS
C = 40
NCH = EPW // C
RPT = 624
CPR = 48
OFFS = sorted(set(list(range(0, C - 15, 16)) + [C - 16]))


def _sc_edge_body(ei_hbm, elt_hbm, ert_hbm, feat_hbm,
                  elmax_hbm, ermax_hbm,
                  acc_hbm, s_hbm,
                  acc_sh, s_sh,
                  idxs, gidxs, gas, gbs, gfs, gfws, wbufs, dscs,
                  mxa, mxb, obuf, sbuf,
                  gsems, ssems, isems, zsem):
    c = lax.axis_index("c")
    sid = lax.axis_index("s")

    rbase = sid * RPT
    zv = jnp.zeros((16,), jnp.float32)
    for r in range(CPR):
        for j in range(F // 16):
            obuf[r, 16 * j:16 * j + 16] = zv
        sbuf[r, :] = zv

    zds = []
    for k in range(RPT // CPR):
        b = rbase + k * CPR
        zds.append(pltpu.async_copy(obuf, acc_sh.at[pl.ds(b, CPR)], zsem))
        zds.append(pltpu.async_copy(sbuf, s_sh.at[pl.ds(b, CPR)], zsem))

    @pl.when(sid == NS - 1)
    def _():
        t = N - NS * RPT
        pltpu.async_copy(obuf.at[pl.ds(0, t)],
                         acc_sh.at[pl.ds(NS * RPT, t)], zsem).wait()
        pltpu.async_copy(sbuf.at[pl.ds(0, t)],
                         s_sh.at[pl.ds(NS * RPT, t)], zsem).wait()

    for d in zds:
        d.wait()
    plsc.subcore_barrier()

    pltpu.sync_copy(elmax_hbm.at[c], mxa)
    pltpu.sync_copy(ermax_hbm.at[c], mxb)
    shiftv = jnp.maximum(mxa[0] + mxb[0], 0.0)
    cnv = jnp.full((16,), c * N, jnp.int32)

    ebase = sid * EPW

    def fire_gathers(s):
        gidx = gidxs.at[s]
        pltpu.async_copy(elt_hbm.at[gidx.at[0]], gas.at[s], gsems.at[s])
        pltpu.async_copy(ert_hbm.at[gidx.at[1]], gbs.at[s], gsems.at[s])
        pltpu.async_copy(feat_hbm.at[gidx.at[0]], gfs.at[s], gsems.at[s])

    def fire_idx(s, cidx):
        base = jnp.minimum(ebase + cidx * C, E - C)
        pltpu.async_copy(ei_hbm.at[c, :, pl.ds(base, C)], idxs.at[s],
                         isems.at[s])

    def wait_gathers(s):
        gidx = gidxs.at[s]
        pltpu.make_async_copy(elt_hbm.at[gidx.at[0]], gas.at[s],
                              gsems.at[s]).wait()
        pltpu.make_async_copy(ert_hbm.at[gidx.at[1]], gbs.at[s],
                              gsems.at[s]).wait()
        pltpu.make_async_copy(feat_hbm.at[gidx.at[0]], gfs.at[s],
                              gsems.at[s]).wait()

    def wait_scatters(s):
        pltpu.make_async_copy(wbufs.at[s], s_sh.at[dscs.at[s].at[0]],
                              ssems.at[s]).wait()
        pltpu.make_async_copy(gfws.at[s], acc_sh.at[dscs.at[s].at[0]],
                              ssems.at[s]).wait()

    def step(s, i, cidx):
        wait_gathers(s)
        for o in OFFS:
            dscs[s, 0, o:o + 16] = idxs[s, 1, o:o + 16]

        @pl.when(i > 0)
        def _():
            wait_scatters(s)

        fire_idx(s, cidx + 2)
        for e in range(C):
            z = gas[s, e] + gbs[s, e]
            z = jnp.maximum(z, z * 0.2)
            w = jnp.exp(z - shiftv)
            wbufs[s, e] = w
            for h in range(NH):
                whv = jnp.full((16,), w[h], jnp.float32)
                sl = slice(16 * h, 16 * h + 16)
                gfws[s, e, sl] = gfs[s, e, sl] * whv
        pltpu.make_async_copy(ei_hbm.at[0, :, pl.ds(0, C)], idxs.at[s],
                              isems.at[s]).wait()
        for r in range(2):
            for o in OFFS:
                gidxs[s, r, o:o + 16] = idxs[s, r, o:o + 16] + cnv
        fire_gathers(s)
        pltpu.async_copy(wbufs.at[s], s_sh.at[dscs.at[s].at[0]],
                         ssems.at[s], add=True)
        pltpu.async_copy(gfws.at[s], acc_sh.at[dscs.at[s].at[0]],
                         ssems.at[s], add=True)

    for s in range(2):
        base = ebase + s * C
        pltpu.sync_copy(ei_hbm.at[c, :, pl.ds(base, C)], idxs.at[s])
        for r in range(2):
            for o in OFFS:
                gidxs[s, r, o:o + 16] = idxs[s, r, o:o + 16] + cnv
        fire_gathers(s)

    def pair_step(i, _):
        step(0, i, 2 * i)
        step(1, i, 2 * i + 1)
        return 0

    lax.fori_loop(0, NCH // 2, pair_step, 0)
    if NCH % 2:
        step(0, jnp.int32(1), NCH - 1)

    for s in range(2):
        wait_gathers(s)
        wait_scatters(s)
    plsc.subcore_barrier()

    ods = []
    for k in range(RPT // CPR):
        b = rbase + k * CPR
        ods.append(pltpu.async_copy(acc_sh.at[pl.ds(b, CPR)],
                                    acc_hbm.at[c, pl.ds(b, CPR)], zsem))
        ods.append(pltpu.async_copy(s_sh.at[pl.ds(b, CPR)],
                                    s_hbm.at[c, pl.ds(b, CPR)], zsem))

    @pl.when(sid == NS - 1)
    def _():
        t = N - NS * RPT
        pltpu.async_copy(acc_sh.at[pl.ds(NS * RPT, t)],
                         acc_hbm.at[c, pl.ds(NS * RPT, t)], zsem).wait()
        pltpu.async_copy(s_sh.at[pl.ds(NS * RPT, t)],
                         s_hbm.at[c, pl.ds(NS * RPT, t)], zsem).wait()

    for d in ods:
        d.wait()


@jax.jit
def _sc_edge(ei2, elt2, ert2, feat2, elmax2, ermax2):
    mesh = plsc.VectorSubcoreMesh(core_axis_name="c", subcore_axis_name="s")
    f = pl.kernel(
        _sc_edge_body,
        mesh=mesh,
        compiler_params=pltpu.CompilerParams(use_tc_tiling_on_sc=False,
                                             needs_layout_passes=False),
        out_type=[
            jax.ShapeDtypeStruct((NG, N, F), jnp.float32),
            jax.ShapeDtypeStruct((NG, N, 16), jnp.float32),
        ],
        scratch_types=[
            pltpu.VMEM_SHARED((N, F), jnp.float32),
            pltpu.VMEM_SHARED((N, 16), jnp.float32),
            pltpu.VMEM((2, 2, C), jnp.int32),
            pltpu.VMEM((2, 2, C), jnp.int32),
            pltpu.VMEM((2, C, 16), jnp.float32),
            pltpu.VMEM((2, C, 16), jnp.float32),
            pltpu.VMEM((2, C, F), jnp.float32),
            pltpu.VMEM((2, C, F), jnp.float32),
            pltpu.VMEM((2, C, 16), jnp.float32),
            pltpu.VMEM((2, 1, C), jnp.int32),
            pltpu.VMEM((1, 16), jnp.float32),
            pltpu.VMEM((1, 16), jnp.float32),
            pltpu.VMEM((CPR, F), jnp.float32),
            pltpu.VMEM((CPR, 16), jnp.float32),
            pltpu.SemaphoreType.DMA((2,)),
            pltpu.SemaphoreType.DMA((2,)),
            pltpu.SemaphoreType.DMA((2,)),
            pltpu.SemaphoreType.DMA,
        ],
    )
    return f(ei2, elt2, ert2, feat2, elmax2, ermax2)


BN = 400
GRID = N // BN


def _prep_common(featb, al, ar, feat_ref, elt_ref, ert_ref,
                 elmax_ref, ermax_ref, i):
    feat_ref[...] = featb[None]
    f3 = featb.reshape(BN, NH, HD)
    el = jnp.sum(f3 * al[None], axis=-1)
    er = jnp.sum(f3 * ar[None], axis=-1)
    zpad = jnp.zeros((BN, 16 - NH), jnp.float32)
    elt_ref[...] = jnp.concatenate([el, zpad], axis=1)[None]
    ert_ref[...] = jnp.concatenate([er, zpad], axis=1)[None]
    zp = jnp.zeros((1, 16 - NH), jnp.float32)
    bm_el = jnp.concatenate([jnp.max(el, axis=0)[None], zp], axis=1)[None]
    bm_er = jnp.concatenate([jnp.max(er, axis=0)[None], zp], axis=1)[None]

    @pl.when(i == 0)
    def _():
        elmax_ref[...] = bm_el
        ermax_ref[...] = bm_er

    @pl.when(i > 0)
    def _():
        elmax_ref[...] = jnp.maximum(elmax_ref[...], bm_el)
        ermax_ref[...] = jnp.maximum(ermax_ref[...], bm_er)


def _tc_prep_body(h_ref, w_ref, al_ref, ar_ref,
                  feat_ref, elt_ref, ert_ref, elmax_ref, ermax_ref):
    i = pl.program_id(1)
    featb = jnp.dot(h_ref[0], w_ref[0], preferred_element_type=jnp.float32)
    _prep_common(featb, al_ref[0], ar_ref[0], feat_ref, elt_ref, ert_ref,
                 elmax_ref, ermax_ref, i)


_PREP_OUT = [
    jax.ShapeDtypeStruct((NG, N, F), jnp.float32),
    jax.ShapeDtypeStruct((NG, N, 16), jnp.float32),
    jax.ShapeDtypeStruct((NG, N, 16), jnp.float32),
    jax.ShapeDtypeStruct((NG, 1, 16), jnp.float32),
    jax.ShapeDtypeStruct((NG, 1, 16), jnp.float32),
]
_PREP_OUT_SPECS = [
    pl.BlockSpec((1, BN, F), lambda g, i: (g, i, 0)),
    pl.BlockSpec((1, BN, 16), lambda g, i: (g, i, 0)),
    pl.BlockSpec((1, BN, 16), lambda g, i: (g, i, 0)),
    pl.BlockSpec((1, 1, 16), lambda g, i: (g, 0, 0)),
    pl.BlockSpec((1, 1, 16), lambda g, i: (g, 0, 0)),
]


@jax.jit
def _tc_prep(h2, w2, al2, ar2):
    return pl.pallas_call(
        _tc_prep_body,
        grid=(NG, GRID),
        in_specs=[
            pl.BlockSpec((1, BN, F), lambda g, i: (g, i, 0)),
            pl.BlockSpec((1, F, F), lambda g, i: (g, 0, 0)),
            pl.BlockSpec((1, NH, HD), lambda g, i: (g, 0, 0)),
            pl.BlockSpec((1, NH, HD), lambda g, i: (g, 0, 0)),
        ],
        out_specs=_PREP_OUT_SPECS,
        out_shape=_PREP_OUT,
    )(h2, w2, al2, ar2)


def _post_block(accb, sb, b):
    sv = sb[:, :NH]
    sexp = jnp.broadcast_to(sv[:, :, None], (BN, NH, HD)).reshape(BN, F)
    x = accb / (sexp + 1e-9) + b
    return SELU_SCALE * jnp.where(x > 0, x, SELU_ALPHA * (jnp.exp(x) - 1.0))


def _tc_mid_body(acc_ref, s_ref, b_ref, w_ref, al_ref, ar_ref,
                 feat_ref, elt_ref, ert_ref, elmax_ref, ermax_ref):
    i = pl.program_id(1)
    hn = _post_block(acc_ref[0], s_ref[0], b_ref[0])
    featb = jnp.dot(hn, w_ref[0], preferred_element_type=jnp.float32)
    _prep_common(featb, al_ref[0], ar_ref[0], feat_ref, elt_ref, ert_ref,
                 elmax_ref, ermax_ref, i)


@jax.jit
def _tc_mid(acc, s, b2, w2, al2, ar2):
    return pl.pallas_call(
        _tc_mid_body,
        grid=(NG, GRID),
        in_specs=[
            pl.BlockSpec((1, BN, F), lambda g, i: (g, i, 0)),
            pl.BlockSpec((1, BN, 16), lambda g, i: (g, i, 0)),
            pl.BlockSpec((1, 1, F), lambda g, i: (g, 0, 0)),
            pl.BlockSpec((1, F, F), lambda g, i: (g, 0, 0)),
            pl.BlockSpec((1, NH, HD), lambda g, i: (g, 0, 0)),
            pl.BlockSpec((1, NH, HD), lambda g, i: (g, 0, 0)),
        ],
        out_specs=_PREP_OUT_SPECS,
        out_shape=_PREP_OUT,
    )(acc, s, b2, w2, al2, ar2)


def _tc_readout_body(acc_ref, s_ref, b_ref, mask_ref, r_ref):
    i = pl.program_id(1)
    hn = _post_block(acc_ref[0], s_ref[0], b_ref[0])
    hm = jnp.mean(hn.reshape(BN, NH, HD), axis=1)
    hm = hm * mask_ref[...]
    part = (jnp.sum(hm, axis=0, keepdims=True) * (1.0 / N))[None]

    @pl.when(i == 0)
    def _():
        r_ref[...] = part

    @pl.when(i > 0)
    def _():
        r_ref[...] = r_ref[...] + part


@jax.jit
def _tc_readout(acc, s, b2, mask2):
    return pl.pallas_call(
        _tc_readout_body,
        grid=(NG, GRID),
        in_specs=[
            pl.BlockSpec((1, BN, F), lambda g, i: (g, i, 0)),
            pl.BlockSpec((1, BN, 16), lambda g, i: (g, i, 0)),
            pl.BlockSpec((1, 1, F), lambda g, i: (g, 0, 0)),
            pl.BlockSpec((BN, 1), lambda g, i: (i, 0)),
        ],
        out_specs=pl.BlockSpec((1, 1, HD), lambda g, i: (g, 0, 0)),
        out_shape=jax.ShapeDtypeStruct((NG, 1, HD), jnp.float32),
    )(acc, s, b2, mask2)


def _tc_combine_body(r_ref, wc_ref, bc_ref, out_ref):
    hav = (r_ref[0] + r_ref[1]) * 0.5
    lg = jnp.dot(hav, wc_ref[...],
                 preferred_element_type=jnp.float32) + bc_ref[...]
    m = jnp.max(lg)
    ex = jnp.exp(lg - m)
    out_ref[...] = ex / jnp.sum(ex)


@jax.jit
def _tc_combine(r, wc, bc2):
    return pl.pallas_call(
        _tc_combine_body,
        out_shape=jax.ShapeDtypeStruct((1, 2), jnp.float32),
    )(r, wc, bc2)


def _stack_layer(pa, pb, key):
    return jnp.stack([pa[key], pb[key]])


def kernel(x_arg, edge_index_arg, x_ast, edge_index_ast, mask, params):
    mask2 = mask.reshape(N, 1)
    la, lb = params["gat_arg"], params["gat_ast"]
    x2 = jnp.stack([x_arg, x_ast])
    ei2 = jnp.stack([edge_index_arg, edge_index_ast])
    w1 = _stack_layer(la[0], lb[0], "W")
    al1 = _stack_layer(la[0], lb[0], "al")
    ar1 = _stack_layer(la[0], lb[0], "ar")
    b1 = _stack_layer(la[0], lb[0], "b").reshape(NG, 1, F)
    w2 = _stack_layer(la[1], lb[1], "W")
    al2 = _stack_layer(la[1], lb[1], "al")
    ar2 = _stack_layer(la[1], lb[1], "ar")
    b2 = _stack_layer(la[1], lb[1], "b").reshape(NG, 1, F)

    feat, elt, ert, elm, erm = _tc_prep(x2, w1, al1, ar1)
    acc, s = _sc_edge(ei2, elt.reshape(NG * N, 16), ert.reshape(NG * N, 16),
                      feat.reshape(NG * N, F), elm, erm)
    feat2, elt2, ert2, elm2, erm2 = _tc_mid(acc, s, b1, w2, al2, ar2)
    acc2, s2 = _sc_edge(ei2, elt2.reshape(NG * N, 16),
                        ert2.reshape(NG * N, 16),
                        feat2.reshape(NG * N, F), elm2, erm2)
    r = _tc_readout(acc2, s2, b2, mask2)
    out = _tc_combine(r, params["Wc"], params["bc"].reshape(1, 2))
    return out.reshape(2)


# speedup vs baseline: 1.1453x; 1.1453x over previous
import functools

import jax
import jax.numpy as jnp
from jax import lax
from jax.experimental import pallas as pl
from jax.experimental.pallas import tpu as pltpu
from jax.experimental.pallas import tpu_sc as plsc

N = 10000
E = 320000
F = 128
NH = 8
HD = 16

SELU_ALPHA = 1.6732632423543772
SELU_SCALE = 1.0507009873554805

NC = 2
NS = 16
NW = NC * NS
EPW = E // NW
C = 40
NCH = EPW // C
RPT = 624
CPR = 48
OFFS = sorted(set(list(range(0, C - 15, 16)) + [C - 16]))


def _sc_edge_body(ei_hbm, elt_hbm, ert_hbm, feat_hbm,
                  elmax_hbm, ermax_hbm,
                  acc_hbm, s_hbm,
                  acc_sh, s_sh,
                  idxs, gas, gbs, gfs, gfws, wbufs, dscs,
                  mxa, mxb, obuf, sbuf,
                  gsems, ssems, isems, zsem):
    c = lax.axis_index("c")
    sid = lax.axis_index("s")
    wid = c * NS + sid

    rbase = sid * RPT
    zv = jnp.zeros((16,), jnp.float32)
    for r in range(CPR):
        for j in range(F // 16):
            obuf[r, 16 * j:16 * j + 16] = zv
        sbuf[r, :] = zv

    zds = []
    for k in range(RPT // CPR):
        b = rbase + k * CPR
        zds.append(pltpu.async_copy(obuf, acc_sh.at[pl.ds(b, CPR)], zsem))
        zds.append(pltpu.async_copy(sbuf, s_sh.at[pl.ds(b, CPR)], zsem))

    @pl.when(sid == NS - 1)
    def _():
        t = N - NS * RPT
        pltpu.async_copy(obuf.at[pl.ds(0, t)],
                         acc_sh.at[pl.ds(NS * RPT, t)], zsem).wait()
        pltpu.async_copy(sbuf.at[pl.ds(0, t)],
                         s_sh.at[pl.ds(NS * RPT, t)], zsem).wait()

    for d in zds:
        d.wait()
    plsc.subcore_barrier()

    pltpu.sync_copy(elmax_hbm, mxa)
    pltpu.sync_copy(ermax_hbm, mxb)
    shiftv = jnp.maximum(mxa[0] + mxb[0], 0.0)

    ebase = wid * EPW

    def fire_gathers(s):
        idx = idxs.at[s]
        pltpu.async_copy(elt_hbm.at[idx.at[0]], gas.at[s], gsems.at[s])
        pltpu.async_copy(ert_hbm.at[idx.at[1]], gbs.at[s], gsems.at[s])
        pltpu.async_copy(feat_hbm.at[idx.at[0]], gfs.at[s], gsems.at[s])

    def fire_idx(s, cidx):
        base = jnp.minimum(ebase + cidx * C, E - C)
        pltpu.async_copy(ei_hbm.at[:, pl.ds(base, C)], idxs.at[s],
                         isems.at[s])

    def wait_gathers(s):
        idx = idxs.at[s]
        pltpu.make_async_copy(elt_hbm.at[idx.at[0]], gas.at[s],
                              gsems.at[s]).wait()
        pltpu.make_async_copy(ert_hbm.at[idx.at[1]], gbs.at[s],
                              gsems.at[s]).wait()
        pltpu.make_async_copy(feat_hbm.at[idx.at[0]], gfs.at[s],
                              gsems.at[s]).wait()

    def wait_scatters(s):
        pltpu.make_async_copy(wbufs.at[s], s_sh.at[dscs.at[s].at[0]],
                              ssems.at[s]).wait()
        pltpu.make_async_copy(gfws.at[s], acc_sh.at[dscs.at[s].at[0]],
                              ssems.at[s]).wait()

    def step(s, i, cidx):
        wait_gathers(s)
        for o in OFFS:
            dscs[s, 0, o:o + 16] = idxs[s, 1, o:o + 16]

        @pl.when(i > 0)
        def _():
            wait_scatters(s)

        fire_idx(s, cidx + 2)
        for e in range(C):
            z = gas[s, e] + gbs[s, e]
            z = jnp.maximum(z, z * 0.2)
            w = jnp.exp(z - shiftv)
            wbufs[s, e] = w
            for h in range(NH):
                whv = jnp.full((16,), w[h], jnp.float32)
                sl = slice(16 * h, 16 * h + 16)
                gfws[s, e, sl] = gfs[s, e, sl] * whv
        pltpu.make_async_copy(ei_hbm.at[:, pl.ds(0, C)], idxs.at[s],
                              isems.at[s]).wait()
        fire_gathers(s)
        pltpu.async_copy(wbufs.at[s], s_sh.at[dscs.at[s].at[0]],
                         ssems.at[s], add=True)
        pltpu.async_copy(gfws.at[s], acc_sh.at[dscs.at[s].at[0]],
                         ssems.at[s], add=True)

    for s in range(2):
        base = ebase + s * C
        pltpu.sync_copy(ei_hbm.at[:, pl.ds(base, C)], idxs.at[s])
        fire_gathers(s)

    def pair_step(i, _):
        step(0, i, 2 * i)
        step(1, i, 2 * i + 1)
        return 0

    lax.fori_loop(0, NCH // 2, pair_step, 0)
    if NCH % 2:
        step(0, jnp.int32(1), NCH - 1)

    for s in range(2):
        wait_gathers(s)
        wait_scatters(s)
    plsc.subcore_barrier()

    ods = []
    for k in range(RPT // CPR):
        b = rbase + k * CPR
        ods.append(pltpu.async_copy(acc_sh.at[pl.ds(b, CPR)],
                                    acc_hbm.at[c, pl.ds(b, CPR)], zsem))
        ods.append(pltpu.async_copy(s_sh.at[pl.ds(b, CPR)],
                                    s_hbm.at[c, pl.ds(b, CPR)], zsem))

    @pl.when(sid == NS - 1)
    def _():
        t = N - NS * RPT
        pltpu.async_copy(acc_sh.at[pl.ds(NS * RPT, t)],
                         acc_hbm.at[c, pl.ds(NS * RPT, t)], zsem).wait()
        pltpu.async_copy(s_sh.at[pl.ds(NS * RPT, t)],
                         s_hbm.at[c, pl.ds(NS * RPT, t)], zsem).wait()

    for d in ods:
        d.wait()


@jax.jit
def _sc_edge(ei, elt, ert, feat, elmax, ermax):
    mesh = plsc.VectorSubcoreMesh(core_axis_name="c", subcore_axis_name="s")
    f = pl.kernel(
        _sc_edge_body,
        mesh=mesh,
        compiler_params=pltpu.CompilerParams(use_tc_tiling_on_sc=False,
                                             needs_layout_passes=False),
        out_type=[
            jax.ShapeDtypeStruct((NC, N, F), jnp.float32),
            jax.ShapeDtypeStruct((NC, N, 16), jnp.float32),
        ],
        scratch_types=[
            pltpu.VMEM_SHARED((N, F), jnp.float32),
            pltpu.VMEM_SHARED((N, 16), jnp.float32),
            pltpu.VMEM((2, 2, C), jnp.int32),
            pltpu.VMEM((2, C, 16), jnp.float32),
            pltpu.VMEM((2, C, 16), jnp.float32),
            pltpu.VMEM((2, C, F), jnp.float32),
            pltpu.VMEM((2, C, F), jnp.float32),
            pltpu.VMEM((2, C, 16), jnp.float32),
            pltpu.VMEM((2, 1, C), jnp.int32),
            pltpu.VMEM((1, 16), jnp.float32),
            pltpu.VMEM((1, 16), jnp.float32),
            pltpu.VMEM((CPR, F), jnp.float32),
            pltpu.VMEM((CPR, 16), jnp.float32),
            pltpu.SemaphoreType.DMA((2,)),
            pltpu.SemaphoreType.DMA((2,)),
            pltpu.SemaphoreType.DMA((2,)),
            pltpu.SemaphoreType.DMA,
        ],
    )
    return f(ei, elt, ert, feat, elmax, ermax)


BN = 400
GRID = N // BN


def _prep_common(featb, al, ar, feat_ref, elt_ref, ert_ref,
                 elmax_ref, ermax_ref, i):
    feat_ref[...] = featb
    f3 = featb.reshape(BN, NH, HD)
    el = jnp.sum(f3 * al[None], axis=-1)
    er = jnp.sum(f3 * ar[None], axis=-1)
    zpad = jnp.zeros((BN, 16 - NH), jnp.float32)
    elt_ref[...] = jnp.concatenate([el, zpad], axis=1)
    ert_ref[...] = jnp.concatenate([er, zpad], axis=1)
    zp = jnp.zeros((1, 16 - NH), jnp.float32)
    bm_el = jnp.concatenate([jnp.max(el, axis=0)[None], zp], axis=1)
    bm_er = jnp.concatenate([jnp.max(er, axis=0)[None], zp], axis=1)

    @pl.when(i == 0)
    def _():
        elmax_ref[...] = bm_el
        ermax_ref[...] = bm_er

    @pl.when(i > 0)
    def _():
        elmax_ref[...] = jnp.maximum(elmax_ref[...], bm_el)
        ermax_ref[...] = jnp.maximum(ermax_ref[...], bm_er)


def _tc_prep_body(h_ref, w_ref, al_ref, ar_ref,
                  feat_ref, elt_ref, ert_ref, elmax_ref, ermax_ref):
    i = pl.program_id(0)
    featb = jnp.dot(h_ref[...], w_ref[...], preferred_element_type=jnp.float32)
    _prep_common(featb, al_ref[...], ar_ref[...], feat_ref, elt_ref, ert_ref,
                 elmax_ref, ermax_ref, i)


_PREP_OUT = [
    jax.ShapeDtypeStruct((N, F), jnp.float32),
    jax.ShapeDtypeStruct((N, 16), jnp.float32),
    jax.ShapeDtypeStruct((N, 16), jnp.float32),
    jax.ShapeDtypeStruct((1, 16), jnp.float32),
    jax.ShapeDtypeStruct((1, 16), jnp.float32),
]
_PREP_OUT_SPECS = [
    pl.BlockSpec((BN, F), lambda i: (i, 0)),
    pl.BlockSpec((BN, 16), lambda i: (i, 0)),
    pl.BlockSpec((BN, 16), lambda i: (i, 0)),
    pl.BlockSpec((1, 16), lambda i: (0, 0)),
    pl.BlockSpec((1, 16), lambda i: (0, 0)),
]


@jax.jit
def _tc_prep(h, w, al, ar):
    return pl.pallas_call(
        _tc_prep_body,
        grid=(GRID,),
        in_specs=[
            pl.BlockSpec((BN, F), lambda i: (i, 0)),
            pl.BlockSpec((F, F), lambda i: (0, 0)),
            pl.BlockSpec((NH, HD), lambda i: (0, 0)),
            pl.BlockSpec((NH, HD), lambda i: (0, 0)),
        ],
        out_specs=_PREP_OUT_SPECS,
        out_shape=_PREP_OUT,
    )(h, w, al, ar)


def _post_block(accb, sb, b):
    accb = accb.astype(jnp.float32)
    a = accb[0] + accb[1]
    sv = sb[0, :, :NH] + sb[1, :, :NH]
    sexp = jnp.broadcast_to(sv[:, :, None], (BN, NH, HD)).reshape(BN, F)
    x = a / (sexp + 1e-9) + b
    return SELU_SCALE * jnp.where(x > 0, x, SELU_ALPHA * (jnp.exp(x) - 1.0))


def _tc_mid_body(acc_ref, s_ref, b_ref, w_ref, al_ref, ar_ref,
                 feat_ref, elt_ref, ert_ref, elmax_ref, ermax_ref):
    i = pl.program_id(0)
    hn = _post_block(acc_ref[...], s_ref[...], b_ref[...])
    featb = jnp.dot(hn, w_ref[...], preferred_element_type=jnp.float32)
    _prep_common(featb, al_ref[...], ar_ref[...], feat_ref, elt_ref, ert_ref,
                 elmax_ref, ermax_ref, i)


@jax.jit
def _tc_mid(acc, s, b, w, al, ar):
    b = b.reshape(1, F)
    return pl.pallas_call(
        _tc_mid_body,
        grid=(GRID,),
        in_specs=[
            pl.BlockSpec((NC, BN, F), lambda i: (0, i, 0)),
            pl.BlockSpec((NC, BN, 16), lambda i: (0, i, 0)),
            pl.BlockSpec((1, F), lambda i: (0, 0)),
            pl.BlockSpec((F, F), lambda i: (0, 0)),
            pl.BlockSpec((NH, HD), lambda i: (0, 0)),
            pl.BlockSpec((NH, HD), lambda i: (0, 0)),
        ],
        out_specs=_PREP_OUT_SPECS,
        out_shape=_PREP_OUT,
    )(acc, s, b, w, al, ar)


def _tc_readout_body(acc_ref, s_ref, b_ref, mask_ref, r_ref):
    i = pl.program_id(0)
    hn = _post_block(acc_ref[...], s_ref[...], b_ref[...])
    hm = jnp.mean(hn.reshape(BN, NH, HD), axis=1)
    hm = hm * mask_ref[...]
    part = jnp.sum(hm, axis=0, keepdims=True) * (1.0 / N)

    @pl.when(i == 0)
    def _():
        r_ref[...] = part

    @pl.when(i > 0)
    def _():
        r_ref[...] = r_ref[...] + part


@jax.jit
def _tc_readout(acc, s, b, mask2):
    b = b.reshape(1, F)
    return pl.pallas_call(
        _tc_readout_body,
        grid=(GRID,),
        in_specs=[
            pl.BlockSpec((NC, BN, F), lambda i: (0, i, 0)),
            pl.BlockSpec((NC, BN, 16), lambda i: (0, i, 0)),
            pl.BlockSpec((1, F), lambda i: (0, 0)),
            pl.BlockSpec((BN, 1), lambda i: (i, 0)),
        ],
        out_specs=pl.BlockSpec((1, HD), lambda i: (0, 0)),
        out_shape=jax.ShapeDtypeStruct((1, HD), jnp.float32),
    )(acc, s, b, mask2)


def _tc_combine_body(r1_ref, r2_ref, wc_ref, bc_ref, out_ref):
    hav = (r1_ref[...] + r2_ref[...]) * 0.5
    lg = jnp.dot(hav, wc_ref[...],
                 preferred_element_type=jnp.float32) + bc_ref[...]
    m = jnp.max(lg)
    ex = jnp.exp(lg - m)
    out_ref[...] = ex / jnp.sum(ex)


@jax.jit
def _tc_combine(r1, r2, wc, bc2):
    return pl.pallas_call(
        _tc_combine_body,
        out_shape=jax.ShapeDtypeStruct((1, 2), jnp.float32),
    )(r1, r2, wc, bc2)


def _run_graph(x, edge_index, layers, mask2):
    p1, p2 = layers
    feat, elt, ert, elm, erm = _tc_prep(x, p1["W"], p1["al"], p1["ar"])
    acc, s = _sc_edge(edge_index, elt, ert, feat, elm, erm)
    feat2, elt2, ert2, elm2, erm2 = _tc_mid(
        acc, s, p1["b"], p2["W"], p2["al"], p2["ar"])
    acc2, s2 = _sc_edge(edge_index, elt2, ert2, feat2, elm2, erm2)
    return _tc_readout(acc2, s2, p2["b"], mask2)


def kernel(x_arg, edge_index_arg, x_ast, edge_index_ast, mask, params):
    mask2 = mask.reshape(N, 1)
    r1 = _run_graph(x_arg, edge_index_arg, params["gat_arg"], mask2)
    r2 = _run_graph(x_ast, edge_index_ast, params["gat_ast"], mask2)
    out = _tc_combine(r1, r2, params["Wc"], params["bc"].reshape(1, 2))
    return out.reshape(2)


# speedup vs baseline: 1.3907x; 1.2142x over previous
import functools

import jax
import jax.numpy as jnp
from jax import lax
from jax.experimental import pallas as pl
from jax.experimental.pallas import tpu as pltpu
from jax.experimental.pallas import tpu_sc as plsc

N = 10000
E = 320000
F = 128
NH = 8
HD = 16

SELU_ALPHA = 1.6732632423543772
SELU_SCALE = 1.0507009873554805

NC = 2
NS = 16
NW = NC * NS
EPW = E // NW
C = 40
NCH = EPW // C
NSET = 3
RPT = 624
CPR = 16
OFFS = sorted(set(list(range(0, C - 15, 16)) + [C - 16]))


def _sc_edge_body(ei_hbm, elt_hbm, ert_hbm, feat_hbm,
                  elmax_hbm, ermax_hbm,
                  acc_hbm, s_hbm,
                  acc_sh, s_sh,
                  idxs, gas, gbs, gfs, gfws, wbufs, dscs,
                  mxa, mxb, obuf, sbuf,
                  gsems, ssems, isems, zsem):
    c = lax.axis_index("c")
    sid = lax.axis_index("s")
    wid = c * NS + sid

    rbase = sid * RPT
    zv = jnp.zeros((16,), jnp.float32)
    for r in range(CPR):
        for j in range(F // 16):
            obuf[r, 16 * j:16 * j + 16] = zv
        sbuf[r, :] = zv

    zds = []
    for k in range(RPT // CPR):
        b = rbase + k * CPR
        zds.append(pltpu.async_copy(obuf, acc_sh.at[pl.ds(b, CPR)], zsem))
        zds.append(pltpu.async_copy(sbuf, s_sh.at[pl.ds(b, CPR)], zsem))

    @pl.when(sid == NS - 1)
    def _():
        t = N - NS * RPT
        pltpu.async_copy(obuf.at[pl.ds(0, t)],
                         acc_sh.at[pl.ds(NS * RPT, t)], zsem).wait()
        pltpu.async_copy(sbuf.at[pl.ds(0, t)],
                         s_sh.at[pl.ds(NS * RPT, t)], zsem).wait()

    for d in zds:
        d.wait()
    plsc.subcore_barrier()

    pltpu.sync_copy(elmax_hbm, mxa)
    pltpu.sync_copy(ermax_hbm, mxb)
    shiftv = jnp.maximum(mxa[0] + mxb[0], 0.0)

    ebase = wid * EPW

    def fire_gathers(s):
        idx = idxs.at[s]
        pltpu.async_copy(elt_hbm.at[idx.at[0]], gas.at[s], gsems.at[s])
        pltpu.async_copy(ert_hbm.at[idx.at[1]], gbs.at[s], gsems.at[s])
        pltpu.async_copy(feat_hbm.at[idx.at[0]], gfs.at[s], gsems.at[s])

    def fire_idx(s, cidx):
        base = jnp.minimum(ebase + cidx * C, E - C)
        pltpu.async_copy(ei_hbm.at[:, pl.ds(base, C)], idxs.at[s],
                         isems.at[s])

    def wait_gathers(s):
        idx = idxs.at[s]
        pltpu.make_async_copy(elt_hbm.at[idx.at[0]], gas.at[s],
                              gsems.at[s]).wait()
        pltpu.make_async_copy(ert_hbm.at[idx.at[1]], gbs.at[s],
                              gsems.at[s]).wait()
        pltpu.make_async_copy(feat_hbm.at[idx.at[0]], gfs.at[s],
                              gsems.at[s]).wait()

    def wait_scatters(s):
        pltpu.make_async_copy(wbufs.at[s], s_sh.at[dscs.at[s].at[0]],
                              ssems.at[s]).wait()
        pltpu.make_async_copy(gfws.at[s], acc_sh.at[dscs.at[s].at[0]],
                              ssems.at[s]).wait()

    def step(s, i, cidx):
        wait_gathers(s)
        for o in OFFS:
            dscs[s, 0, o:o + 16] = idxs[s, 1, o:o + 16]

        @pl.when(i > 0)
        def _():
            wait_scatters(s)

        fire_idx(s, cidx + NSET)
        for e in range(C):
            z = gas[s, e] + gbs[s, e]
            z = jnp.maximum(z, z * 0.2)
            w = jnp.exp(z - shiftv)
            wbufs[s, e] = w
            for h in range(NH):
                whv = jnp.full((16,), w[h], jnp.float32)
                sl = slice(16 * h, 16 * h + 16)
                gfws[s, e, sl] = gfs[s, e, sl] * whv
        pltpu.make_async_copy(ei_hbm.at[:, pl.ds(0, C)], idxs.at[s],
                              isems.at[s]).wait()
        fire_gathers(s)
        pltpu.async_copy(wbufs.at[s], s_sh.at[dscs.at[s].at[0]],
                         ssems.at[s], add=True)
        pltpu.async_copy(gfws.at[s], acc_sh.at[dscs.at[s].at[0]],
                         ssems.at[s], add=True)

    for s in range(NSET):
        base = ebase + s * C
        pltpu.sync_copy(ei_hbm.at[:, pl.ds(base, C)], idxs.at[s])
        fire_gathers(s)

    def rot_step(i, _):
        for s in range(NSET):
            step(s, i, NSET * i + s)
        return 0

    lax.fori_loop(0, NCH // NSET, rot_step, 0)
    for r in range(NCH % NSET):
        step(r, jnp.int32(1), (NCH // NSET) * NSET + r)

    for s in range(NSET):
        wait_gathers(s)
        wait_scatters(s)
    plsc.subcore_barrier()

    ods = []
    for k in range(RPT // CPR):
        b = rbase + k * CPR
        ods.append(pltpu.async_copy(acc_sh.at[pl.ds(b, CPR)],
                                    acc_hbm.at[c, pl.ds(b, CPR)], zsem))
        ods.append(pltpu.async_copy(s_sh.at[pl.ds(b, CPR)],
                                    s_hbm.at[c, pl.ds(b, CPR)], zsem))

    @pl.when(sid == NS - 1)
    def _():
        t = N - NS * RPT
        pltpu.async_copy(acc_sh.at[pl.ds(NS * RPT, t)],
                         acc_hbm.at[c, pl.ds(NS * RPT, t)], zsem).wait()
        pltpu.async_copy(s_sh.at[pl.ds(NS * RPT, t)],
                         s_hbm.at[c, pl.ds(NS * RPT, t)], zsem).wait()

    for d in ods:
        d.wait()


@jax.jit
def _sc_edge(ei, elt, ert, feat, elmax, ermax):
    mesh = plsc.VectorSubcoreMesh(core_axis_name="c", subcore_axis_name="s")
    f = pl.kernel(
        _sc_edge_body,
        mesh=mesh,
        compiler_params=pltpu.CompilerParams(use_tc_tiling_on_sc=False,
                                             needs_layout_passes=False),
        out_type=[
            jax.ShapeDtypeStruct((NC, N, F), jnp.float32),
            jax.ShapeDtypeStruct((NC, N, 16), jnp.float32),
        ],
        scratch_types=[
            pltpu.VMEM_SHARED((N, F), jnp.float32),
            pltpu.VMEM_SHARED((N, 16), jnp.float32),
            pltpu.VMEM((NSET, 2, C), jnp.int32),
            pltpu.VMEM((NSET, C, 16), jnp.float32),
            pltpu.VMEM((NSET, C, 16), jnp.float32),
            pltpu.VMEM((NSET, C, F), jnp.float32),
            pltpu.VMEM((NSET, C, F), jnp.float32),
            pltpu.VMEM((NSET, C, 16), jnp.float32),
            pltpu.VMEM((NSET, 1, C), jnp.int32),
            pltpu.VMEM((1, 16), jnp.float32),
            pltpu.VMEM((1, 16), jnp.float32),
            pltpu.VMEM((CPR, F), jnp.float32),
            pltpu.VMEM((CPR, 16), jnp.float32),
            pltpu.SemaphoreType.DMA((NSET,)),
            pltpu.SemaphoreType.DMA((NSET,)),
            pltpu.SemaphoreType.DMA((NSET,)),
            pltpu.SemaphoreType.DMA,
        ],
    )
    return f(ei, elt, ert, feat, elmax, ermax)


BN = 400
GRID = N // BN


def _prep_common(featb, al, ar, feat_ref, elt_ref, ert_ref,
                 elmax_ref, ermax_ref, i):
    feat_ref[...] = featb
    f3 = featb.reshape(BN, NH, HD)
    el = jnp.sum(f3 * al[None], axis=-1)
    er = jnp.sum(f3 * ar[None], axis=-1)
    zpad = jnp.zeros((BN, 16 - NH), jnp.float32)
    elt_ref[...] = jnp.concatenate([el, zpad], axis=1)
    ert_ref[...] = jnp.concatenate([er, zpad], axis=1)
    zp = jnp.zeros((1, 16 - NH), jnp.float32)
    bm_el = jnp.concatenate([jnp.max(el, axis=0)[None], zp], axis=1)
    bm_er = jnp.concatenate([jnp.max(er, axis=0)[None], zp], axis=1)

    @pl.when(i == 0)
    def _():
        elmax_ref[...] = bm_el
        ermax_ref[...] = bm_er

    @pl.when(i > 0)
    def _():
        elmax_ref[...] = jnp.maximum(elmax_ref[...], bm_el)
        ermax_ref[...] = jnp.maximum(ermax_ref[...], bm_er)


def _tc_prep_body(h_ref, w_ref, al_ref, ar_ref,
                  feat_ref, elt_ref, ert_ref, elmax_ref, ermax_ref):
    i = pl.program_id(0)
    featb = jnp.dot(h_ref[...], w_ref[...], preferred_element_type=jnp.float32)
    _prep_common(featb, al_ref[...], ar_ref[...], feat_ref, elt_ref, ert_ref,
                 elmax_ref, ermax_ref, i)


_PREP_OUT = [
    jax.ShapeDtypeStruct((N, F), jnp.float32),
    jax.ShapeDtypeStruct((N, 16), jnp.float32),
    jax.ShapeDtypeStruct((N, 16), jnp.float32),
    jax.ShapeDtypeStruct((1, 16), jnp.float32),
    jax.ShapeDtypeStruct((1, 16), jnp.float32),
]
_PREP_OUT_SPECS = [
    pl.BlockSpec((BN, F), lambda i: (i, 0)),
    pl.BlockSpec((BN, 16), lambda i: (i, 0)),
    pl.BlockSpec((BN, 16), lambda i: (i, 0)),
    pl.BlockSpec((1, 16), lambda i: (0, 0)),
    pl.BlockSpec((1, 16), lambda i: (0, 0)),
]


@jax.jit
def _tc_prep(h, w, al, ar):
    return pl.pallas_call(
        _tc_prep_body,
        grid=(GRID,),
        in_specs=[
            pl.BlockSpec((BN, F), lambda i: (i, 0)),
            pl.BlockSpec((F, F), lambda i: (0, 0)),
            pl.BlockSpec((NH, HD), lambda i: (0, 0)),
            pl.BlockSpec((NH, HD), lambda i: (0, 0)),
        ],
        out_specs=_PREP_OUT_SPECS,
        out_shape=_PREP_OUT,
    )(h, w, al, ar)


def _post_block(accb, sb, b):
    accb = accb.astype(jnp.float32)
    a = accb[0] + accb[1]
    sv = sb[0, :, :NH] + sb[1, :, :NH]
    sexp = jnp.broadcast_to(sv[:, :, None], (BN, NH, HD)).reshape(BN, F)
    x = a / (sexp + 1e-9) + b
    return SELU_SCALE * jnp.where(x > 0, x, SELU_ALPHA * (jnp.exp(x) - 1.0))


def _tc_mid_body(acc_ref, s_ref, b_ref, w_ref, al_ref, ar_ref,
                 feat_ref, elt_ref, ert_ref, elmax_ref, ermax_ref):
    i = pl.program_id(0)
    hn = _post_block(acc_ref[...], s_ref[...], b_ref[...])
    featb = jnp.dot(hn, w_ref[...], preferred_element_type=jnp.float32)
    _prep_common(featb, al_ref[...], ar_ref[...], feat_ref, elt_ref, ert_ref,
                 elmax_ref, ermax_ref, i)


@jax.jit
def _tc_mid(acc, s, b, w, al, ar):
    b = b.reshape(1, F)
    return pl.pallas_call(
        _tc_mid_body,
        grid=(GRID,),
        in_specs=[
            pl.BlockSpec((NC, BN, F), lambda i: (0, i, 0)),
            pl.BlockSpec((NC, BN, 16), lambda i: (0, i, 0)),
            pl.BlockSpec((1, F), lambda i: (0, 0)),
            pl.BlockSpec((F, F), lambda i: (0, 0)),
            pl.BlockSpec((NH, HD), lambda i: (0, 0)),
            pl.BlockSpec((NH, HD), lambda i: (0, 0)),
        ],
        out_specs=_PREP_OUT_SPECS,
        out_shape=_PREP_OUT,
    )(acc, s, b, w, al, ar)


def _tc_readout_body(acc_ref, s_ref, b_ref, mask_ref, r_ref):
    i = pl.program_id(0)
    hn = _post_block(acc_ref[...], s_ref[...], b_ref[...])
    hm = jnp.mean(hn.reshape(BN, NH, HD), axis=1)
    hm = hm * mask_ref[...]
    part = jnp.sum(hm, axis=0, keepdims=True) * (1.0 / N)

    @pl.when(i == 0)
    def _():
        r_ref[...] = part

    @pl.when(i > 0)
    def _():
        r_ref[...] = r_ref[...] + part


@jax.jit
def _tc_readout(acc, s, b, mask2):
    b = b.reshape(1, F)
    return pl.pallas_call(
        _tc_readout_body,
        grid=(GRID,),
        in_specs=[
            pl.BlockSpec((NC, BN, F), lambda i: (0, i, 0)),
            pl.BlockSpec((NC, BN, 16), lambda i: (0, i, 0)),
            pl.BlockSpec((1, F), lambda i: (0, 0)),
            pl.BlockSpec((BN, 1), lambda i: (i, 0)),
        ],
        out_specs=pl.BlockSpec((1, HD), lambda i: (0, 0)),
        out_shape=jax.ShapeDtypeStruct((1, HD), jnp.float32),
    )(acc, s, b, mask2)


def _tc_combine_body(r1_ref, r2_ref, wc_ref, bc_ref, out_ref):
    hav = (r1_ref[...] + r2_ref[...]) * 0.5
    lg = jnp.dot(hav, wc_ref[...],
                 preferred_element_type=jnp.float32) + bc_ref[...]
    m = jnp.max(lg)
    ex = jnp.exp(lg - m)
    out_ref[...] = ex / jnp.sum(ex)


@jax.jit
def _tc_combine(r1, r2, wc, bc2):
    return pl.pallas_call(
        _tc_combine_body,
        out_shape=jax.ShapeDtypeStruct((1, 2), jnp.float32),
    )(r1, r2, wc, bc2)


def _run_graph(x, edge_index, layers, mask2):
    p1, p2 = layers
    feat, elt, ert, elm, erm = _tc_prep(x, p1["W"], p1["al"], p1["ar"])
    acc, s = _sc_edge(edge_index, elt, ert, feat, elm, erm)
    feat2, elt2, ert2, elm2, erm2 = _tc_mid(
        acc, s, p1["b"], p2["W"], p2["al"], p2["ar"])
    acc2, s2 = _sc_edge(edge_index, elt2, ert2, feat2, elm2, erm2)
    return _tc_readout(acc2, s2, p2["b"], mask2)


def kernel(x_arg, edge_index_arg, x_ast, edge_index_ast, mask, params):
    mask2 = mask.reshape(N, 1)
    r1 = _run_graph(x_arg, edge_index_arg, params["gat_arg"], mask2)
    r2 = _run_graph(x_ast, edge_index_ast, params["gat_ast"], mask2)
    out = _tc_combine(r1, r2, params["Wc"], params["bc"].reshape(1, 2))
    return out.reshape(2)


# speedup vs baseline: 1.4007x; 1.0072x over previous
import functools

import jax
import jax.numpy as jnp
from jax import lax
from jax.experimental import pallas as pl
from jax.experimental.pallas import tpu as pltpu
from jax.experimental.pallas import tpu_sc as plsc

N = 10000
E = 320000
F = 128
NH = 8
HD = 16

SELU_ALPHA = 1.6732632423543772
SELU_SCALE = 1.0507009873554805

NC = 2
NS = 16
NW = NC * NS
EPW = E // NW
C = 40
NCH = EPW // C
NSET = 3
RPT = 624
CPR = 16
OFFS = sorted(set(list(range(0, C - 15, 16)) + [C - 16]))


def _sc_edge_body(ei_hbm, elt_hbm, ert_hbm, feat_hbm,
                  elmax_hbm, ermax_hbm,
                  acc_hbm, s_hbm,
                  acc_sh, s_sh,
                  idxs, gas, gbs, gfs, gfws, wbufs, dscs,
                  mxa, mxb, obuf, sbuf,
                  gsems, ssems, isems, zsem):
    c = lax.axis_index("c")
    sid = lax.axis_index("s")
    wid = c * NS + sid

    rbase = sid * RPT
    zv = jnp.zeros((16,), jnp.float32)
    for r in range(CPR):
        for j in range(F // 16):
            obuf[r, 16 * j:16 * j + 16] = zv
        sbuf[r, :] = zv

    zds = []
    for k in range(RPT // CPR):
        b = rbase + k * CPR
        zds.append(pltpu.async_copy(obuf, acc_sh.at[pl.ds(b, CPR)], zsem))
        zds.append(pltpu.async_copy(sbuf, s_sh.at[pl.ds(b, CPR)], zsem))

    @pl.when(sid == NS - 1)
    def _():
        t = N - NS * RPT
        pltpu.async_copy(obuf.at[pl.ds(0, t)],
                         acc_sh.at[pl.ds(NS * RPT, t)], zsem).wait()
        pltpu.async_copy(sbuf.at[pl.ds(0, t)],
                         s_sh.at[pl.ds(NS * RPT, t)], zsem).wait()

    for d in zds:
        d.wait()
    plsc.subcore_barrier()

    pltpu.sync_copy(elmax_hbm, mxa)
    pltpu.sync_copy(ermax_hbm, mxb)
    shiftv = jnp.maximum(mxa[0] + mxb[0], 0.0)

    ebase = wid * EPW

    def fire_gathers(s):
        idx = idxs.at[s]
        pltpu.async_copy(elt_hbm.at[idx.at[0]], gas.at[s], gsems.at[s])
        pltpu.async_copy(ert_hbm.at[idx.at[1]], gbs.at[s], gsems.at[s])
        pltpu.async_copy(feat_hbm.at[idx.at[0]], gfs.at[s], gsems.at[s])

    def fire_idx(s, cidx):
        base = jnp.minimum(ebase + cidx * C, E - C)
        pltpu.async_copy(ei_hbm.at[:, pl.ds(base, C)], idxs.at[s],
                         isems.at[s])

    def wait_gathers(s):
        idx = idxs.at[s]
        pltpu.make_async_copy(elt_hbm.at[idx.at[0]], gas.at[s],
                              gsems.at[s]).wait()
        pltpu.make_async_copy(ert_hbm.at[idx.at[1]], gbs.at[s],
                              gsems.at[s]).wait()
        pltpu.make_async_copy(feat_hbm.at[idx.at[0]], gfs.at[s],
                              gsems.at[s]).wait()

    def wait_scatters(s):
        pltpu.make_async_copy(wbufs.at[s], s_sh.at[dscs.at[s].at[0]],
                              ssems.at[s]).wait()
        pltpu.make_async_copy(gfws.at[s], acc_sh.at[dscs.at[s].at[0]],
                              ssems.at[s]).wait()

    def step(s, i, cidx):
        wait_gathers(s)
        for o in OFFS:
            dscs[s, 0, o:o + 16] = idxs[s, 1, o:o + 16]

        @pl.when(i > 0)
        def _():
            wait_scatters(s)

        fire_idx(s, cidx + NSET)
        for e in range(C):
            z = gas[s, e] + gbs[s, e]
            z = jnp.maximum(z, z * 0.2)
            w = jnp.exp(z - shiftv)
            wbufs[s, e] = w
            for h in range(NH):
                whv = jnp.full((16,), w[h], jnp.float32)
                sl = slice(16 * h, 16 * h + 16)
                gfws[s, e, sl] = gfs[s, e, sl] * whv
        pltpu.async_copy(wbufs.at[s], s_sh.at[dscs.at[s].at[0]],
                         ssems.at[s], add=True)
        pltpu.async_copy(gfws.at[s], acc_sh.at[dscs.at[s].at[0]],
                         ssems.at[s], add=True)
        pltpu.make_async_copy(ei_hbm.at[:, pl.ds(0, C)], idxs.at[s],
                              isems.at[s]).wait()
        fire_gathers(s)

    for s in range(NSET):
        base = ebase + s * C
        pltpu.sync_copy(ei_hbm.at[:, pl.ds(base, C)], idxs.at[s])
        fire_gathers(s)

    def rot_step(i, _):
        for s in range(NSET):
            step(s, i, NSET * i + s)
        return 0

    lax.fori_loop(0, NCH // NSET, rot_step, 0)
    for r in range(NCH % NSET):
        step(r, jnp.int32(1), (NCH // NSET) * NSET + r)

    for s in range(NSET):
        wait_gathers(s)
        wait_scatters(s)
    plsc.subcore_barrier()

    ods = []
    for k in range(RPT // CPR):
        b = rbase + k * CPR
        ods.append(pltpu.async_copy(acc_sh.at[pl.ds(b, CPR)],
                                    acc_hbm.at[c, pl.ds(b, CPR)], zsem))
        ods.append(pltpu.async_copy(s_sh.at[pl.ds(b, CPR)],
                                    s_hbm.at[c, pl.ds(b, CPR)], zsem))

    @pl.when(sid == NS - 1)
    def _():
        t = N - NS * RPT
        pltpu.async_copy(acc_sh.at[pl.ds(NS * RPT, t)],
                         acc_hbm.at[c, pl.ds(NS * RPT, t)], zsem).wait()
        pltpu.async_copy(s_sh.at[pl.ds(NS * RPT, t)],
                         s_hbm.at[c, pl.ds(NS * RPT, t)], zsem).wait()

    for d in ods:
        d.wait()


@jax.jit
def _sc_edge(ei, elt, ert, feat, elmax, ermax):
    mesh = plsc.VectorSubcoreMesh(core_axis_name="c", subcore_axis_name="s")
    f = pl.kernel(
        _sc_edge_body,
        mesh=mesh,
        compiler_params=pltpu.CompilerParams(use_tc_tiling_on_sc=False,
                                             needs_layout_passes=False),
        out_type=[
            jax.ShapeDtypeStruct((NC, N, F), jnp.float32),
            jax.ShapeDtypeStruct((NC, N, 16), jnp.float32),
        ],
        scratch_types=[
            pltpu.VMEM_SHARED((N, F), jnp.float32),
            pltpu.VMEM_SHARED((N, 16), jnp.float32),
            pltpu.VMEM((NSET, 2, C), jnp.int32),
            pltpu.VMEM((NSET, C, 16), jnp.float32),
            pltpu.VMEM((NSET, C, 16), jnp.float32),
            pltpu.VMEM((NSET, C, F), jnp.float32),
            pltpu.VMEM((NSET, C, F), jnp.float32),
            pltpu.VMEM((NSET, C, 16), jnp.float32),
            pltpu.VMEM((NSET, 1, C), jnp.int32),
            pltpu.VMEM((1, 16), jnp.float32),
            pltpu.VMEM((1, 16), jnp.float32),
            pltpu.VMEM((CPR, F), jnp.float32),
            pltpu.VMEM((CPR, 16), jnp.float32),
            pltpu.SemaphoreType.DMA((NSET,)),
            pltpu.SemaphoreType.DMA((NSET,)),
            pltpu.SemaphoreType.DMA((NSET,)),
            pltpu.SemaphoreType.DMA,
        ],
    )
    return f(ei, elt, ert, feat, elmax, ermax)


BN = 400
GRID = N // BN


def _prep_common(featb, al, ar, feat_ref, elt_ref, ert_ref,
                 elmax_ref, ermax_ref, i):
    feat_ref[...] = featb
    f3 = featb.reshape(BN, NH, HD)
    el = jnp.sum(f3 * al[None], axis=-1)
    er = jnp.sum(f3 * ar[None], axis=-1)
    zpad = jnp.zeros((BN, 16 - NH), jnp.float32)
    elt_ref[...] = jnp.concatenate([el, zpad], axis=1)
    ert_ref[...] = jnp.concatenate([er, zpad], axis=1)
    zp = jnp.zeros((1, 16 - NH), jnp.float32)
    bm_el = jnp.concatenate([jnp.max(el, axis=0)[None], zp], axis=1)
    bm_er = jnp.concatenate([jnp.max(er, axis=0)[None], zp], axis=1)

    @pl.when(i == 0)
    def _():
        elmax_ref[...] = bm_el
        ermax_ref[...] = bm_er

    @pl.when(i > 0)
    def _():
        elmax_ref[...] = jnp.maximum(elmax_ref[...], bm_el)
        ermax_ref[...] = jnp.maximum(ermax_ref[...], bm_er)


def _tc_prep_body(h_ref, w_ref, al_ref, ar_ref,
                  feat_ref, elt_ref, ert_ref, elmax_ref, ermax_ref):
    i = pl.program_id(0)
    featb = jnp.dot(h_ref[...], w_ref[...], preferred_element_type=jnp.float32)
    _prep_common(featb, al_ref[...], ar_ref[...], feat_ref, elt_ref, ert_ref,
                 elmax_ref, ermax_ref, i)


_PREP_OUT = [
    jax.ShapeDtypeStruct((N, F), jnp.float32),
    jax.ShapeDtypeStruct((N, 16), jnp.float32),
    jax.ShapeDtypeStruct((N, 16), jnp.float32),
    jax.ShapeDtypeStruct((1, 16), jnp.float32),
    jax.ShapeDtypeStruct((1, 16), jnp.float32),
]
_PREP_OUT_SPECS = [
    pl.BlockSpec((BN, F), lambda i: (i, 0)),
    pl.BlockSpec((BN, 16), lambda i: (i, 0)),
    pl.BlockSpec((BN, 16), lambda i: (i, 0)),
    pl.BlockSpec((1, 16), lambda i: (0, 0)),
    pl.BlockSpec((1, 16), lambda i: (0, 0)),
]


@jax.jit
def _tc_prep(h, w, al, ar):
    return pl.pallas_call(
        _tc_prep_body,
        grid=(GRID,),
        in_specs=[
            pl.BlockSpec((BN, F), lambda i: (i, 0)),
            pl.BlockSpec((F, F), lambda i: (0, 0)),
            pl.BlockSpec((NH, HD), lambda i: (0, 0)),
            pl.BlockSpec((NH, HD), lambda i: (0, 0)),
        ],
        out_specs=_PREP_OUT_SPECS,
        out_shape=_PREP_OUT,
    )(h, w, al, ar)


def _post_block(accb, sb, b):
    accb = accb.astype(jnp.float32)
    a = accb[0] + accb[1]
    sv = sb[0, :, :NH] + sb[1, :, :NH]
    sexp = jnp.broadcast_to(sv[:, :, None], (BN, NH, HD)).reshape(BN, F)
    x = a / (sexp + 1e-9) + b
    return SELU_SCALE * jnp.where(x > 0, x, SELU_ALPHA * (jnp.exp(x) - 1.0))


def _tc_mid_body(acc_ref, s_ref, b_ref, w_ref, al_ref, ar_ref,
                 feat_ref, elt_ref, ert_ref, elmax_ref, ermax_ref):
    i = pl.program_id(0)
    hn = _post_block(acc_ref[...], s_ref[...], b_ref[...])
    featb = jnp.dot(hn, w_ref[...], preferred_element_type=jnp.float32)
    _prep_common(featb, al_ref[...], ar_ref[...], feat_ref, elt_ref, ert_ref,
                 elmax_ref, ermax_ref, i)


@jax.jit
def _tc_mid(acc, s, b, w, al, ar):
    b = b.reshape(1, F)
    return pl.pallas_call(
        _tc_mid_body,
        grid=(GRID,),
        in_specs=[
            pl.BlockSpec((NC, BN, F), lambda i: (0, i, 0)),
            pl.BlockSpec((NC, BN, 16), lambda i: (0, i, 0)),
            pl.BlockSpec((1, F), lambda i: (0, 0)),
            pl.BlockSpec((F, F), lambda i: (0, 0)),
            pl.BlockSpec((NH, HD), lambda i: (0, 0)),
            pl.BlockSpec((NH, HD), lambda i: (0, 0)),
        ],
        out_specs=_PREP_OUT_SPECS,
        out_shape=_PREP_OUT,
    )(acc, s, b, w, al, ar)


def _tc_readout_body(acc_ref, s_ref, b_ref, mask_ref, r_ref):
    i = pl.program_id(0)
    hn = _post_block(acc_ref[...], s_ref[...], b_ref[...])
    hm = jnp.mean(hn.reshape(BN, NH, HD), axis=1)
    hm = hm * mask_ref[...]
    part = jnp.sum(hm, axis=0, keepdims=True) * (1.0 / N)

    @pl.when(i == 0)
    def _():
        r_ref[...] = part

    @pl.when(i > 0)
    def _():
        r_ref[...] = r_ref[...] + part


@jax.jit
def _tc_readout(acc, s, b, mask2):
    b = b.reshape(1, F)
    return pl.pallas_call(
        _tc_readout_body,
        grid=(GRID,),
        in_specs=[
            pl.BlockSpec((NC, BN, F), lambda i: (0, i, 0)),
            pl.BlockSpec((NC, BN, 16), lambda i: (0, i, 0)),
            pl.BlockSpec((1, F), lambda i: (0, 0)),
            pl.BlockSpec((BN, 1), lambda i: (i, 0)),
        ],
        out_specs=pl.BlockSpec((1, HD), lambda i: (0, 0)),
        out_shape=jax.ShapeDtypeStruct((1, HD), jnp.float32),
    )(acc, s, b, mask2)


def _tc_combine_body(r1_ref, r2_ref, wc_ref, bc_ref, out_ref):
    hav = (r1_ref[...] + r2_ref[...]) * 0.5
    lg = jnp.dot(hav, wc_ref[...],
                 preferred_element_type=jnp.float32) + bc_ref[...]
    m = jnp.max(lg)
    ex = jnp.exp(lg - m)
    out_ref[...] = ex / jnp.sum(ex)


@jax.jit
def _tc_combine(r1, r2, wc, bc2):
    return pl.pallas_call(
        _tc_combine_body,
        out_shape=jax.ShapeDtypeStruct((1, 2), jnp.float32),
    )(r1, r2, wc, bc2)


def _run_graph(x, edge_index, layers, mask2):
    p1, p2 = layers
    feat, elt, ert, elm, erm = _tc_prep(x, p1["W"], p1["al"], p1["ar"])
    acc, s = _sc_edge(edge_index, elt, ert, feat, elm, erm)
    feat2, elt2, ert2, elm2, erm2 = _tc_mid(
        acc, s, p1["b"], p2["W"], p2["al"], p2["ar"])
    acc2, s2 = _sc_edge(edge_index, elt2, ert2, feat2, elm2, erm2)
    return _tc_readout(acc2, s2, p2["b"], mask2)


def kernel(x_arg, edge_index_arg, x_ast, edge_index_ast, mask, params):
    mask2 = mask.reshape(N, 1)
    r1 = _run_graph(x_arg, edge_index_arg, params["gat_arg"], mask2)
    r2 = _run_graph(x_ast, edge_index_ast, params["gat_ast"], mask2)
    out = _tc_combine(r1, r2, params["Wc"], params["bc"].reshape(1, 2))
    return out.reshape(2)
